# baseline (speedup 1.0000x reference)
import jax
import jax.numpy as jnp
from jax.experimental import pallas as pl

MARGIN_ = 0.3
BLK_ = 128


def _loss_body(a_ref, t_row_ref, t_col_ref, out_ref):
    n = a_ref.shape[0]
    nblk = n // BLK_
    one = jnp.bfloat16(1.0)
    zero = jnp.bfloat16(0.0)
    margin = jnp.bfloat16(MARGIN_)

    a_bf = a_ref[...].astype(jnp.bfloat16)
    t_row = t_row_ref[...].astype(jnp.bfloat16)
    t_col = t_col_ref[...].astype(jnp.bfloat16)

    parts = []
    for i in range(nblk):
        r0 = i * BLK_
        a_i = a_bf[r0:r0 + BLK_, :]
        t_i = t_row[r0:r0 + BLK_, :]
        a_w = a_bf[r0:, :]
        t_w = t_col[:, r0:]
        sim = jax.lax.dot_general(
            a_i, a_w,
            dimension_numbers=(((1,), (1,)), ((), ())),
            preferred_element_type=jnp.float32,
        ).astype(jnp.bfloat16)
        same = t_i == t_w
        pos = jnp.maximum(one - sim, zero)
        neg = jnp.where(sim > margin, sim, zero)
        contrib = jnp.where(same, pos, neg)
        red = contrib
        while red.shape[0] > 8:
            h = red.shape[0] // 2
            red = red[:h, :] + red[h:, :]
        s_all = jnp.sum(red.astype(jnp.float32))
        s_diag = jnp.sum(red[:, :BLK_].astype(jnp.float32))
        parts.append(2.0 * s_all - s_diag)

    total = jnp.sum(jnp.stack(parts))
    out_ref[...] = (total * (1.0 / n))[None, None]


def kernel(inputs, targets):
    n, d = inputs.shape
    t_row = targets.reshape(n, 1)
    t_col = targets.reshape(1, n)

    out = pl.pallas_call(
        _loss_body,
        out_shape=jax.ShapeDtypeStruct((1, 1), jnp.float32),
    )(inputs, t_row, t_col)
    return out[0, 0]


# speedup vs baseline: 1.0008x; 1.0008x over previous
import jax
import jax.numpy as jnp
from jax.experimental import pallas as pl
from jax.experimental.pallas import tpu as pltpu

MARGIN_ = 0.3
BLK_ = 256


def _loss_body(a_ref, t_row_ref, t_col_ref, out_ref):
    n = a_ref.shape[0]
    nblk = n // BLK_
    one = jnp.bfloat16(1.0)
    zero = jnp.bfloat16(0.0)
    margin = jnp.bfloat16(MARGIN_)

    a_bf = a_ref[...].astype(jnp.bfloat16)
    t_row = t_row_ref[...].astype(jnp.bfloat16)
    t_col = t_col_ref[...].astype(jnp.bfloat16)

    parts = []
    for i in range(nblk):
        r0 = i * BLK_
        a_i = a_bf[r0:r0 + BLK_, :]
        t_i = t_row[r0:r0 + BLK_, :]
        a_w = a_bf[r0:, :]
        t_w = t_col[:, r0:]
        sim = jax.lax.dot_general(
            a_i, a_w,
            dimension_numbers=(((1,), (1,)), ((), ())),
            preferred_element_type=jnp.float32,
        ).astype(jnp.bfloat16)
        same = t_i == t_w
        pos = jnp.maximum(one - sim, zero)
        neg = jnp.where(sim > margin, sim, zero)
        contrib = jnp.where(same, pos, neg)
        red = contrib
        while red.shape[0] > 8:
            h = red.shape[0] // 2
            red = red[:h, :] + red[h:, :]
        s_all = jnp.sum(red.astype(jnp.float32))
        s_diag = jnp.sum(red[:, :BLK_].astype(jnp.float32))
        parts.append(2.0 * s_all - s_diag)

    total = jnp.sum(jnp.stack(parts))
    out_ref[0] = total * (1.0 / n)


def kernel(inputs, targets):
    n, d = inputs.shape
    t_row = targets.reshape(n, 1)
    t_col = targets.reshape(1, n)

    out = pl.pallas_call(
        _loss_body,
        out_specs=pl.BlockSpec(memory_space=pltpu.SMEM),
        out_shape=jax.ShapeDtypeStruct((1,), jnp.float32),
    )(inputs, t_row, t_col)
    return out[0]


# speedup vs baseline: 1.0102x; 1.0095x over previous
import jax
import jax.numpy as jnp
from jax.experimental import pallas as pl

MARGIN_ = 0.3
BLK_ = 256


def _loss_body(a_ref, t_row_ref, t_col_ref, out_ref):
    n = a_ref.shape[0]
    nblk = n // BLK_
    one = jnp.bfloat16(1.0)
    zero = jnp.bfloat16(0.0)
    margin = jnp.bfloat16(MARGIN_)

    a_bf = a_ref[...].astype(jnp.bfloat16)
    t_row = t_row_ref[...].astype(jnp.bfloat16)
    t_col = t_col_ref[...].astype(jnp.bfloat16)

    parts = []
    for i in range(nblk):
        r0 = i * BLK_
        a_i = a_bf[r0:r0 + BLK_, :]
        t_i = t_row[r0:r0 + BLK_, :]
        a_w = a_bf[r0:, :]
        t_w = t_col[:, r0:]
        sim = jax.lax.dot_general(
            a_i, a_w,
            dimension_numbers=(((1,), (1,)), ((), ())),
            preferred_element_type=jnp.float32,
        ).astype(jnp.bfloat16)
        same = t_i == t_w
        pos = jnp.maximum(one - sim, zero)
        neg = jnp.where(sim > margin, sim, zero)
        contrib = jnp.where(same, pos, neg)
        red = contrib
        while red.shape[0] > 8:
            h = red.shape[0] // 2
            red = red[:h, :] + red[h:, :]
        s_all = jnp.sum(red.astype(jnp.float32))
        s_diag = jnp.sum(red[:, :BLK_].astype(jnp.float32))
        parts.append(2.0 * s_all - s_diag)

    total = jnp.sum(jnp.stack(parts))
    out_ref[...] = (total * (1.0 / n))[None, None]


def kernel(inputs, targets):
    n, d = inputs.shape
    t_row = targets.reshape(n, 1)
    t_col = targets.reshape(1, n)

    out = pl.pallas_call(
        _loss_body,
        out_shape=jax.ShapeDtypeStruct((1, 1), jnp.float32),
    )(inputs, t_row, t_col)
    return out[0, 0]
